# baseline (speedup 1.0000x reference)
import jax
import jax.numpy as jnp
from jax import lax
from jax.experimental import pallas as pl
from jax.experimental.pallas import tpu as pltpu
from jax.experimental.pallas import tpu_sc as plsc

_ENT_NUM = 1000000
_REL_NUM = 1000
_HIDDEN = 64
_BATCH = 4096
_SEQ = 8192
_MARGIN = 1.0

_NC = 2
_NS = 16
_NW = _NC * _NS
_PAIRS = _BATCH // _NW
_L = 16
_GROUPS = _PAIRS // _L
_UNROLL = 4
_W2 = 2 * _HIDDEN


def _rsqrt(x):
    i = plsc.bitcast(x, jnp.int32)
    i = jnp.int32(0x5F3759DF) - (i >> 1)
    y = plsc.bitcast(i, jnp.float32)
    for _ in range(3):
        y = y * (1.5 - 0.5 * x * y * y)
    return y


def _scores(H, T, RN, s_idx, oh, ot):
    zeros = jnp.zeros((_L,), jnp.float32)

    def body(db, carry):
        nn, hn, tn, uu, un = carry
        d0 = db * _UNROLL
        for du in range(_UNROLL):
            dd = jnp.full((_L,), d0 + du, jnp.int32)
            h = plsc.load_gather(H, [s_idx, dd + oh])
            t = plsc.load_gather(T, [s_idx, dd + ot])
            r = plsc.load_gather(RN, [s_idx, dd])
            n = plsc.load_gather(RN, [s_idx, dd + _HIDDEN])
            u = h + r - t
            nn = nn + n * n
            hn = hn + h * n
            tn = tn + t * n
            uu = uu + u * u
            un = un + u * n
        return (nn, hn, tn, uu, un)

    nn, hn, tn, uu, un = lax.fori_loop(
        0, _HIDDEN // _UNROLL, body, (zeros, zeros, zeros, zeros, zeros))
    inv = jnp.minimum(_rsqrt(jnp.maximum(nn, 1e-30)), 1e12)
    alpha = (hn - tn) * inv * inv
    d2 = uu - 2.0 * alpha * un + alpha * alpha * nn
    d2 = jnp.maximum(d2, 0.0)
    return d2 * _rsqrt(jnp.maximum(d2, 1e-30))


def _body(ent2, rn, mh, oh, mt, ot, br, out,
          imh_p, imh_n, imt_p, imt_n, ir_p, ir_n,
          ioh_p, ioh_n, iot_p, iot_n,
          H_p, T_p, H_n, T_n, RN_p, RN_n,
          loss_v, sem):
    w = lax.axis_index("c") * _NS + lax.axis_index("s")
    base_p = w * _PAIRS
    base_n = _BATCH + base_p

    pltpu.sync_copy(mh.at[pl.ds(base_p, _PAIRS)], imh_p)
    pltpu.sync_copy(mh.at[pl.ds(base_n, _PAIRS)], imh_n)
    pltpu.sync_copy(mt.at[pl.ds(base_p, _PAIRS)], imt_p)
    pltpu.sync_copy(mt.at[pl.ds(base_n, _PAIRS)], imt_n)
    pltpu.sync_copy(br.at[pl.ds(base_p, _PAIRS)], ir_p)
    pltpu.sync_copy(br.at[pl.ds(base_n, _PAIRS)], ir_n)
    pltpu.sync_copy(oh.at[pl.ds(base_p, _PAIRS)], ioh_p)
    pltpu.sync_copy(oh.at[pl.ds(base_n, _PAIRS)], ioh_n)
    pltpu.sync_copy(ot.at[pl.ds(base_p, _PAIRS)], iot_p)
    pltpu.sync_copy(ot.at[pl.ds(base_n, _PAIRS)], iot_n)

    copies = [
        pltpu.async_copy(ent2.at[imh_p], H_p, sem),
        pltpu.async_copy(ent2.at[imt_p], T_p, sem),
        pltpu.async_copy(ent2.at[imh_n], H_n, sem),
        pltpu.async_copy(ent2.at[imt_n], T_n, sem),
        pltpu.async_copy(rn.at[ir_p], RN_p, sem),
        pltpu.async_copy(rn.at[ir_n], RN_n, sem),
    ]
    for c in copies:
        c.wait()

    lane = lax.iota(jnp.int32, _L)

    def gbody(g, acc):
        s_idx = g * _L + lane
        ds = pl.ds(g * _L, _L)
        sp = _scores(H_p, T_p, RN_p, s_idx, ioh_p[ds], iot_p[ds])
        sn = _scores(H_n, T_n, RN_n, s_idx, ioh_n[ds], iot_n[ds])
        return acc + jnp.maximum(sp - sn + _MARGIN, 0.0)

    acc = lax.fori_loop(0, _GROUPS, gbody, jnp.zeros((_L,), jnp.float32))
    loss_v[...] = acc
    pltpu.sync_copy(loss_v, out.at[w])


def kernel(ent_embeddings, rel_embeddings, norm_vector, batch_h, batch_t, batch_r):
    ent2 = ent_embeddings.reshape(_ENT_NUM // 2, _W2)
    rn = jnp.concatenate([rel_embeddings, norm_vector], axis=1)
    bh = batch_h.astype(jnp.int32)
    bt = batch_t.astype(jnp.int32)
    mh = bh >> 1
    oh = (bh & 1) << 6
    mt = bt >> 1
    ot = (bt & 1) << 6
    mesh = plsc.VectorSubcoreMesh(core_axis_name="c", subcore_axis_name="s")
    f = pl.kernel(
        _body,
        out_type=jax.ShapeDtypeStruct((_NW, _L), jnp.float32),
        mesh=mesh,
        compiler_params=pltpu.CompilerParams(needs_layout_passes=False),
        scratch_types=[
            pltpu.VMEM((_PAIRS,), jnp.int32),
            pltpu.VMEM((_PAIRS,), jnp.int32),
            pltpu.VMEM((_PAIRS,), jnp.int32),
            pltpu.VMEM((_PAIRS,), jnp.int32),
            pltpu.VMEM((_PAIRS,), jnp.int32),
            pltpu.VMEM((_PAIRS,), jnp.int32),
            pltpu.VMEM((_PAIRS,), jnp.int32),
            pltpu.VMEM((_PAIRS,), jnp.int32),
            pltpu.VMEM((_PAIRS,), jnp.int32),
            pltpu.VMEM((_PAIRS,), jnp.int32),
            pltpu.VMEM((_PAIRS, _W2), jnp.float32),
            pltpu.VMEM((_PAIRS, _W2), jnp.float32),
            pltpu.VMEM((_PAIRS, _W2), jnp.float32),
            pltpu.VMEM((_PAIRS, _W2), jnp.float32),
            pltpu.VMEM((_PAIRS, _W2), jnp.float32),
            pltpu.VMEM((_PAIRS, _W2), jnp.float32),
            pltpu.VMEM((_L,), jnp.float32),
            pltpu.SemaphoreType.DMA,
        ],
    )
    partials = f(ent2, rn, mh, oh, mt, ot, batch_r.astype(jnp.int32))
    return jnp.sum(partials)


# speedup vs baseline: 1.6445x; 1.6445x over previous
import jax
import jax.numpy as jnp
from jax import lax
from jax.experimental import pallas as pl
from jax.experimental.pallas import tpu as pltpu
from jax.experimental.pallas import tpu_sc as plsc

_ENT_NUM = 1000000
_REL_NUM = 1000
_HIDDEN = 64
_BATCH = 4096
_SEQ = 8192
_MARGIN = 1.0

_NC = 2
_NS = 16
_NW = _NC * _NS
_PAIRS = _BATCH // _NW
_L = 16
_GROUPS = _PAIRS // _L
_UNROLL = 4
_ROWS = 4 * _PAIRS


def _rsqrt(x):
    i = plsc.bitcast(x, jnp.int32)
    i = jnp.int32(0x5F3759DF) - (i >> 1)
    y = plsc.bitcast(i, jnp.float32)
    for _ in range(3):
        y = y * (1.5 - 0.5 * x * y * y)
    return y


def _scores(Hall, RN, hbase, tbase, s_idx):
    zeros = jnp.zeros((_L,), jnp.float32)

    def body(db, carry):
        nn, hn, tn, uu, un = carry
        d0 = db * _UNROLL
        for du in range(_UNROLL):
            dd = jnp.full((_L,), d0 + du, jnp.int32)
            h = plsc.load_gather(Hall, [hbase + s_idx, dd])
            t = plsc.load_gather(Hall, [tbase + s_idx, dd])
            r = plsc.load_gather(RN, [s_idx, dd])
            n = plsc.load_gather(RN, [s_idx, dd + _HIDDEN])
            u = h + r - t
            nn = nn + n * n
            hn = hn + h * n
            tn = tn + t * n
            uu = uu + u * u
            un = un + u * n
        return (nn, hn, tn, uu, un)

    nn, hn, tn, uu, un = lax.fori_loop(
        0, _HIDDEN // _UNROLL, body, (zeros, zeros, zeros, zeros, zeros))
    inv = jnp.minimum(_rsqrt(jnp.maximum(nn, 1e-30)), 1e12)
    alpha = (hn - tn) * inv * inv
    d2 = uu - 2.0 * alpha * un + alpha * alpha * nn
    d2 = jnp.maximum(d2, 0.0)
    return d2 * _rsqrt(jnp.maximum(d2, 1e-30))


def _body(ent, rn, bh, bt, br, out,
          ir_p, ir_n, ih_p, it_p, ih_n, it_n,
          Hall, RN_p, RN_n,
          loss_v, sem_rn, sem):
    w = lax.axis_index("c") * _NS + lax.axis_index("s")
    base_p = w * _PAIRS
    base_n = _BATCH + base_p

    pltpu.sync_copy(br.at[pl.ds(base_p, _PAIRS)], ir_p)
    pltpu.sync_copy(br.at[pl.ds(base_n, _PAIRS)], ir_n)
    rn_copies = [
        pltpu.async_copy(rn.at[ir_p], RN_p, sem_rn),
        pltpu.async_copy(rn.at[ir_n], RN_n, sem_rn),
    ]

    pltpu.sync_copy(bh.at[pl.ds(base_p, _PAIRS)], ih_p)
    pltpu.sync_copy(bt.at[pl.ds(base_p, _PAIRS)], it_p)
    pltpu.sync_copy(bh.at[pl.ds(base_n, _PAIRS)], ih_n)
    pltpu.sync_copy(bt.at[pl.ds(base_n, _PAIRS)], it_n)

    def enq(g, carry):
        ds = pl.ds(g * _L, _L)
        vh_p = ih_p[ds]
        vt_p = it_p[ds]
        vh_n = ih_n[ds]
        vt_n = it_n[ds]
        for j in range(_L):
            k = g * _L + j
            pltpu.async_copy(
                ent.at[pl.ds(vh_p[j], 1)], Hall.at[pl.ds(k, 1)], sem)
            pltpu.async_copy(
                ent.at[pl.ds(vt_p[j], 1)], Hall.at[pl.ds(_PAIRS + k, 1)], sem)
            pltpu.async_copy(
                ent.at[pl.ds(vh_n[j], 1)],
                Hall.at[pl.ds(2 * _PAIRS + k, 1)], sem)
            pltpu.async_copy(
                ent.at[pl.ds(vt_n[j], 1)],
                Hall.at[pl.ds(3 * _PAIRS + k, 1)], sem)
        return carry

    lax.fori_loop(0, _GROUPS, enq, 0)

    pltpu.make_async_copy(ent.at[pl.ds(0, _ROWS)], Hall, sem).wait()
    for c in rn_copies:
        c.wait()

    lane = lax.iota(jnp.int32, _L)

    def gbody(g, acc):
        s_idx = g * _L + lane
        sp = _scores(Hall, RN_p, 0, _PAIRS, s_idx)
        sn = _scores(Hall, RN_n, 2 * _PAIRS, 3 * _PAIRS, s_idx)
        return acc + jnp.maximum(sp - sn + _MARGIN, 0.0)

    acc = lax.fori_loop(0, _GROUPS, gbody, jnp.zeros((_L,), jnp.float32))
    loss_v[...] = acc
    pltpu.sync_copy(loss_v, out.at[w])


def kernel(ent_embeddings, rel_embeddings, norm_vector, batch_h, batch_t, batch_r):
    rn = jnp.concatenate([rel_embeddings, norm_vector], axis=1)
    mesh = plsc.VectorSubcoreMesh(core_axis_name="c", subcore_axis_name="s")
    f = pl.kernel(
        _body,
        out_type=jax.ShapeDtypeStruct((_NW, _L), jnp.float32),
        mesh=mesh,
        compiler_params=pltpu.CompilerParams(needs_layout_passes=False),
        scratch_types=[
            pltpu.VMEM((_PAIRS,), jnp.int32),
            pltpu.VMEM((_PAIRS,), jnp.int32),
            pltpu.VMEM((_PAIRS,), jnp.int32),
            pltpu.VMEM((_PAIRS,), jnp.int32),
            pltpu.VMEM((_PAIRS,), jnp.int32),
            pltpu.VMEM((_PAIRS,), jnp.int32),
            pltpu.VMEM((_ROWS, _HIDDEN), jnp.float32),
            pltpu.VMEM((_PAIRS, 2 * _HIDDEN), jnp.float32),
            pltpu.VMEM((_PAIRS, 2 * _HIDDEN), jnp.float32),
            pltpu.VMEM((_L,), jnp.float32),
            pltpu.SemaphoreType.DMA,
            pltpu.SemaphoreType.DMA,
        ],
    )
    partials = f(ent_embeddings, rn, batch_h.astype(jnp.int32),
                 batch_t.astype(jnp.int32), batch_r.astype(jnp.int32))
    return jnp.sum(partials)
